# baseline (speedup 1.0000x reference)
import functools

import jax
import jax.numpy as jnp
from jax.experimental import pallas as pl

POS_DIM = 3
EPS = 1e-5
SIGMA = 0.5


def _compw_body(ew_ref, kw_ref, out_ref):
    out_ref[0] = jnp.dot(ew_ref[...], kw_ref[0],
                         preferred_element_type=jnp.float32)


def _make_comp_w(weights, kw):
    n, c = weights.shape
    k = kw.shape[0]
    return pl.pallas_call(
        _compw_body,
        grid=(k,),
        in_specs=[
            pl.BlockSpec((n, c), lambda i: (0, 0)),
            pl.BlockSpec((1, c, c), lambda i: (i, 0, 0)),
        ],
        out_specs=pl.BlockSpec((1, n, c), lambda i: (i, 0, 0)),
        out_shape=jax.ShapeDtypeStruct((k, n, c), jnp.float32),
    )(weights, kw)


def _sample_body(qpos_ref, qb_ref, cpos_ref, cb_ref, cw_ref, kpos_ref,
                 out_ref, *, inv2s2):
    j = pl.program_id(1)
    kk = pl.program_id(2)

    @pl.when(jnp.logical_and(j == 0, kk == 0))
    def _():
        out_ref[...] = jnp.zeros_like(out_ref)

    zp = qpos_ref[...]
    ep = cpos_ref[...] + kpos_ref[0, 0, :][None, :]
    d2 = (jnp.sum(zp * zp, axis=1)[:, None]
          + jnp.sum(ep * ep, axis=1)[None, :]
          - 2.0 * jnp.dot(zp, ep.T, preferred_element_type=jnp.float32))
    kern = jnp.exp(-d2 * inv2s2)
    zb = qb_ref[0]
    eb = cb_ref[0]
    mask = zb.T == eb
    kern = jnp.where(mask, kern, 0.0)
    out_ref[...] += jnp.dot(kern, cw_ref[0],
                            preferred_element_type=jnp.float32)


def _sample(q_pos, q_batch, c_pos, c_batch, comp_w, kpos, sigma,
            bz=256, be=256):
    nq = q_pos.shape[0]
    nc = c_pos.shape[0]
    k, _, c = comp_w.shape
    gi, gj = nq // bz, nc // be
    qb = q_batch.reshape(gi, 1, bz)
    cb = c_batch.reshape(gj, 1, be)
    kpos3 = kpos.reshape(k, 1, POS_DIM)
    return pl.pallas_call(
        functools.partial(_sample_body, inv2s2=1.0 / (2.0 * sigma * sigma)),
        grid=(gi, gj, k),
        in_specs=[
            pl.BlockSpec((bz, POS_DIM), lambda i, j, kk: (i, 0)),
            pl.BlockSpec((1, 1, bz), lambda i, j, kk: (i, 0, 0)),
            pl.BlockSpec((be, POS_DIM), lambda i, j, kk: (j, 0)),
            pl.BlockSpec((1, 1, be), lambda i, j, kk: (j, 0, 0)),
            pl.BlockSpec((1, be, c), lambda i, j, kk: (kk, j, 0)),
            pl.BlockSpec((1, 1, POS_DIM), lambda i, j, kk: (kk, 0, 0)),
        ],
        out_specs=pl.BlockSpec((bz, c), lambda i, j, kk: (i, 0)),
        out_shape=jax.ShapeDtypeStruct((nq, c), jnp.float32),
    )(q_pos, qb, c_pos, cb, comp_w, kpos3)


def _bnadd_body(x_ref, g_ref, b_ref, base_ref, out_ref):
    x = x_ref[...]
    x = jnp.where(x >= 0, x, 0.01 * x)
    m = jnp.mean(x, axis=0, keepdims=True)
    v = jnp.mean((x - m) ** 2, axis=0, keepdims=True)
    out_ref[...] = (base_ref[...]
                    + (x - m) * jax.lax.rsqrt(v + EPS) * g_ref[...]
                    + b_ref[...])


def _bnadd(x, gamma, beta, base):
    c = x.shape[-1]
    return pl.pallas_call(
        _bnadd_body,
        out_shape=jax.ShapeDtypeStruct(x.shape, jnp.float32),
    )(x, gamma.reshape(1, c), beta.reshape(1, c), base)


def _mlp_body(zw_ref, zpos_ref, w1_ref, b1_ref, g_ref, bt_ref,
              w2p_ref, w2w_ref, b2p_ref, b2w_ref, opos_ref, ow_ref):
    zw = zw_ref[...]
    h = jnp.dot(zw, w1_ref[...], preferred_element_type=jnp.float32)
    h = h + b1_ref[...]
    h = jnp.where(h >= 0, h, 0.01 * h)
    m = jnp.mean(h, axis=0, keepdims=True)
    v = jnp.mean((h - m) ** 2, axis=0, keepdims=True)
    h = (h - m) * jax.lax.rsqrt(v + EPS) * g_ref[...] + bt_ref[...]
    dpos = jnp.dot(h, w2p_ref[...], preferred_element_type=jnp.float32)
    dpos = dpos + b2p_ref[...]
    dw = jnp.dot(h, w2w_ref[...], preferred_element_type=jnp.float32)
    dw = dw + b2w_ref[...]
    opos_ref[...] = zpos_ref[...] + dpos[:, :POS_DIM]
    ow_ref[...] = zw + dw


def kernel(z_positions, z_weights, z_batch, e_positions, e_weights, e_batch,
           cross_kpos, cross_kw, norm_cross_gamma, norm_cross_beta,
           self_kpos, self_kw, norm_self_gamma, norm_self_beta,
           mlp_w1, mlp_b1, mlp_bn_gamma, mlp_bn_beta, mlp_w2, mlp_b2):
    nz, c = z_weights.shape
    c_mlp = mlp_w1.shape[1]

    cw1 = _make_comp_w(e_weights, cross_kw)
    s1 = _sample(z_positions, z_batch, e_positions, e_batch, cw1,
                 cross_kpos, SIGMA)
    zw = _bnadd(s1, norm_cross_gamma, norm_cross_beta, z_weights)

    cw2 = _make_comp_w(zw, self_kw)
    s2 = _sample(z_positions, z_batch, z_positions, z_batch, cw2,
                 self_kpos, SIGMA)
    zw2 = _bnadd(s2, norm_self_gamma, norm_self_beta, zw)

    w2_pos = jnp.zeros((c_mlp, c), jnp.float32).at[:, :POS_DIM].set(
        mlp_w2[:, :POS_DIM])
    b2_pos = jnp.zeros((1, c), jnp.float32).at[0, :POS_DIM].set(
        mlp_b2[:POS_DIM])
    w2_w = mlp_w2[:, POS_DIM:]
    b2_w = mlp_b2[POS_DIM:].reshape(1, c)

    out_pos, out_w = pl.pallas_call(
        _mlp_body,
        out_shape=(
            jax.ShapeDtypeStruct((nz, POS_DIM), jnp.float32),
            jax.ShapeDtypeStruct((nz, c), jnp.float32),
        ),
    )(zw2, z_positions, mlp_w1, mlp_b1.reshape(1, c_mlp),
      mlp_bn_gamma.reshape(1, c_mlp), mlp_bn_beta.reshape(1, c_mlp),
      w2_pos, w2_w, b2_pos, b2_w)
    return out_pos, out_w


# speedup vs baseline: 6.0178x; 6.0178x over previous
import functools

import jax
import jax.numpy as jnp
from jax.experimental import pallas as pl
from jax.experimental.pallas import tpu as pltpu

POS_DIM = 3
EPS = 1e-5
SIGMA = 0.5


def _compw_body(ew_ref, kw_ref, out_ref):
    out_ref[0] = jnp.dot(ew_ref[...], kw_ref[0],
                         preferred_element_type=jnp.float32)


def _make_comp_w(weights, kw):
    n, c = weights.shape
    k = kw.shape[0]
    return pl.pallas_call(
        _compw_body,
        grid=(k,),
        in_specs=[
            pl.BlockSpec((n, c), lambda i: (0, 0)),
            pl.BlockSpec((1, c, c), lambda i: (i, 0, 0)),
        ],
        out_specs=pl.BlockSpec((1, n, c), lambda i: (i, 0, 0)),
        out_shape=jax.ShapeDtypeStruct((k, n, c), jnp.float32),
    )(weights, kw)


def _sample_body(active_ref, jeff_ref, qpos_ref, qb_ref, cpos_ref, cb_ref,
                 cw_ref, kpos_ref, out_ref, *, inv2s2, k):
    i = pl.program_id(0)
    j = pl.program_id(1)

    @pl.when(j == 0)
    def _():
        out_ref[...] = jnp.zeros_like(out_ref)

    @pl.when(active_ref[i, j] != 0)
    def _():
        zp = qpos_ref[...]
        ep = cpos_ref[...]
        kp = kpos_ref[:, 0, :]
        zb = qb_ref[0]
        eb = cb_ref[0]
        mask = zb.T == eb
        z2 = jnp.sum(zp * zp, axis=1)[:, None]
        e2 = jnp.sum(ep * ep, axis=1)[None, :]
        base = (z2 + e2
                - 2.0 * jnp.dot(zp, ep.T, preferred_element_type=jnp.float32))
        k0 = jnp.where(mask, jnp.exp(-base * inv2s2), 0.0)
        zt = jnp.dot(zp, kp.T, preferred_element_type=jnp.float32)
        et = jnp.dot(kp, ep.T, preferred_element_type=jnp.float32)
        t2 = jnp.sum(kp * kp, axis=1)[:, None]
        a = jnp.exp((2.0 * inv2s2) * zt)
        b = jnp.exp(-inv2s2 * (2.0 * et + t2))
        kern = jnp.concatenate(
            [k0 * a[:, kk:kk + 1] * b[kk:kk + 1, :] for kk in range(k)],
            axis=1)
        cw = cw_ref[...].reshape(k * cw_ref.shape[1], cw_ref.shape[2])
        out_ref[...] += jnp.dot(kern, cw, preferred_element_type=jnp.float32)


def _block_meta(q_batch, c_batch, bz, be):
    gi = q_batch.shape[0] // bz
    gj = c_batch.shape[0] // be
    qb = q_batch.reshape(gi, bz)
    cb = c_batch.reshape(gj, be)
    qmin, qmax = qb[:, 0], qb[:, -1]
    cmin, cmax = cb[:, 0], cb[:, -1]
    active = ((cmin[None, :] <= qmax[:, None])
              & (qmin[:, None] <= cmax[None, :])).astype(jnp.int32)
    idx = jnp.where(active == 1, jnp.arange(gj, dtype=jnp.int32)[None, :], -1)
    jeff = jnp.maximum(jax.lax.cummax(idx, axis=1), 0).astype(jnp.int32)
    return active, jeff


def _sample(q_pos, q_batch, c_pos, c_batch, comp_w, kpos, sigma,
            bz=256, be=256):
    nq = q_pos.shape[0]
    nc = c_pos.shape[0]
    k, _, c = comp_w.shape
    gi, gj = nq // bz, nc // be
    active, jeff = _block_meta(q_batch, c_batch, bz, be)
    qb = q_batch.reshape(gi, 1, bz)
    cb = c_batch.reshape(gj, 1, be)
    kpos3 = kpos.reshape(k, 1, POS_DIM)
    grid_spec = pltpu.PrefetchScalarGridSpec(
        num_scalar_prefetch=2,
        grid=(gi, gj),
        in_specs=[
            pl.BlockSpec((bz, POS_DIM), lambda i, j, act, jef: (i, 0)),
            pl.BlockSpec((1, 1, bz), lambda i, j, act, jef: (i, 0, 0)),
            pl.BlockSpec((be, POS_DIM),
                         lambda i, j, act, jef: (jef[i, j], 0)),
            pl.BlockSpec((1, 1, be),
                         lambda i, j, act, jef: (jef[i, j], 0, 0)),
            pl.BlockSpec((k, be, c),
                         lambda i, j, act, jef: (0, jef[i, j], 0)),
            pl.BlockSpec((k, 1, POS_DIM), lambda i, j, act, jef: (0, 0, 0)),
        ],
        out_specs=pl.BlockSpec((bz, c), lambda i, j, act, jef: (i, 0)),
    )
    return pl.pallas_call(
        functools.partial(_sample_body, inv2s2=1.0 / (2.0 * sigma * sigma),
                          k=k),
        grid_spec=grid_spec,
        out_shape=jax.ShapeDtypeStruct((nq, c), jnp.float32),
    )(active, jeff, q_pos, qb, c_pos, cb, comp_w, kpos3)


def _bnadd_body(x_ref, g_ref, b_ref, base_ref, out_ref):
    x = x_ref[...]
    x = jnp.where(x >= 0, x, 0.01 * x)
    m = jnp.mean(x, axis=0, keepdims=True)
    v = jnp.mean((x - m) ** 2, axis=0, keepdims=True)
    out_ref[...] = (base_ref[...]
                    + (x - m) * jax.lax.rsqrt(v + EPS) * g_ref[...]
                    + b_ref[...])


def _bnadd(x, gamma, beta, base):
    c = x.shape[-1]
    return pl.pallas_call(
        _bnadd_body,
        out_shape=jax.ShapeDtypeStruct(x.shape, jnp.float32),
    )(x, gamma.reshape(1, c), beta.reshape(1, c), base)


def _mlp_body(zw_ref, zpos_ref, w1_ref, b1_ref, g_ref, bt_ref,
              w2p_ref, w2w_ref, b2p_ref, b2w_ref, opos_ref, ow_ref):
    zw = zw_ref[...]
    h = jnp.dot(zw, w1_ref[...], preferred_element_type=jnp.float32)
    h = h + b1_ref[...]
    h = jnp.where(h >= 0, h, 0.01 * h)
    m = jnp.mean(h, axis=0, keepdims=True)
    v = jnp.mean((h - m) ** 2, axis=0, keepdims=True)
    h = (h - m) * jax.lax.rsqrt(v + EPS) * g_ref[...] + bt_ref[...]
    dpos = jnp.dot(h, w2p_ref[...], preferred_element_type=jnp.float32)
    dpos = dpos + b2p_ref[...]
    dw = jnp.dot(h, w2w_ref[...], preferred_element_type=jnp.float32)
    dw = dw + b2w_ref[...]
    opos_ref[...] = zpos_ref[...] + dpos[:, :POS_DIM]
    ow_ref[...] = zw + dw


def kernel(z_positions, z_weights, z_batch, e_positions, e_weights, e_batch,
           cross_kpos, cross_kw, norm_cross_gamma, norm_cross_beta,
           self_kpos, self_kw, norm_self_gamma, norm_self_beta,
           mlp_w1, mlp_b1, mlp_bn_gamma, mlp_bn_beta, mlp_w2, mlp_b2):
    nz, c = z_weights.shape
    c_mlp = mlp_w1.shape[1]

    cw1 = _make_comp_w(e_weights, cross_kw)
    s1 = _sample(z_positions, z_batch, e_positions, e_batch, cw1,
                 cross_kpos, SIGMA)
    zw = _bnadd(s1, norm_cross_gamma, norm_cross_beta, z_weights)

    cw2 = _make_comp_w(zw, self_kw)
    s2 = _sample(z_positions, z_batch, z_positions, z_batch, cw2,
                 self_kpos, SIGMA)
    zw2 = _bnadd(s2, norm_self_gamma, norm_self_beta, zw)

    w2_pos = jnp.zeros((c_mlp, c), jnp.float32).at[:, :POS_DIM].set(
        mlp_w2[:, :POS_DIM])
    b2_pos = jnp.zeros((1, c), jnp.float32).at[0, :POS_DIM].set(
        mlp_b2[:POS_DIM])
    w2_w = mlp_w2[:, POS_DIM:]
    b2_w = mlp_b2[POS_DIM:].reshape(1, c)

    out_pos, out_w = pl.pallas_call(
        _mlp_body,
        out_shape=(
            jax.ShapeDtypeStruct((nz, POS_DIM), jnp.float32),
            jax.ShapeDtypeStruct((nz, c), jnp.float32),
        ),
    )(zw2, z_positions, mlp_w1, mlp_b1.reshape(1, c_mlp),
      mlp_bn_gamma.reshape(1, c_mlp), mlp_bn_beta.reshape(1, c_mlp),
      w2_pos, w2_w, b2_pos, b2_w)
    return out_pos, out_w
